# baseline (speedup 1.0000x reference)
import functools

import jax
import jax.numpy as jnp
from jax import lax
from jax.experimental import pallas as pl
from jax.experimental.pallas import tpu as pltpu
from jax.experimental.pallas import tpu_sc as plsc

_N = 10000
_NNZ = 320000
_M = 64000
_B = 128
_D = 128
_L = 3

_NC = 2
_NS = 16
_NW = _NC * _NS

_CH = 8
_MC = 8000
_ACC1 = 8064
_CAP = 2048
_EPP = _NNZ // _NW

_ACC2 = 10112
_PAD2 = 10112

_ZR = 64


def _sc_mesh():
    return plsc.VectorSubcoreMesh(core_axis_name="c", subcore_axis_name="s",
                                  num_cores=_NC, num_subcores=_NS)


@functools.partial(
    pl.kernel,
    out_type=[jax.ShapeDtypeStruct((_CH * _NW * _CAP,), jnp.int32),
              jax.ShapeDtypeStruct((_CH * _NW * _CAP,), jnp.int32),
              jax.ShapeDtypeStruct((_NW * 16,), jnp.int32)],
    mesh=_sc_mesh(),
    compiler_params=pltpu.CompilerParams(needs_layout_passes=False),
    scratch_types=[
        pltpu.VMEM((_EPP,), jnp.int32),
        pltpu.VMEM((_EPP,), jnp.int32),
        pltpu.VMEM((_CH, _CAP), jnp.int32),
        pltpu.VMEM((_CH, _CAP), jnp.int32),
        pltpu.VMEM((16,), jnp.int32),
    ],
)
def _prep(v_hbm, e_hbm, dumv_hbm, dumd_hbm, lv_hbm, ld_hbm, cnt_hbm,
          vsl, esl, vstage, dstage, cbuf):
    c = lax.axis_index("c")
    s = lax.axis_index("s")
    w = c * _NS + s
    pltpu.sync_copy(v_hbm.at[pl.ds(w * _EPP, _EPP)], vsl)
    pltpu.sync_copy(e_hbm.at[pl.ds(w * _EPP, _EPP)], esl)
    for k in range(_CH):
        pltpu.sync_copy(dumv_hbm, vstage.at[k])
        pltpu.sync_copy(dumd_hbm, dstage.at[k])
    iota = lax.broadcasted_iota(jnp.int32, (16,), 0)

    def grp(g, cur):
        off = g * 16
        e16 = esl[pl.ds(off, 16)]
        v16 = vsl[pl.ds(off, 16)]
        c16 = lax.div(e16, jnp.int32(_MC))
        d16 = e16 - c16 * _MC
        out = []
        for k in range(_CH):
            m = c16 == jnp.int32(k)
            cs = plsc.cumsum(jnp.where(m, jnp.int32(1), jnp.int32(0)))
            pos = cur[k] + cs - 1
            mm = m & (pos < _CAP)
            kvec = jnp.full((16,), k, jnp.int32)
            plsc.store_scatter(vstage, [kvec, pos], v16, mask=mm)
            plsc.store_scatter(dstage, [kvec, pos], d16, mask=mm)
            out.append(cur[k] + plsc.all_reduce_population_count(m))
        return tuple(out)

    cur = lax.fori_loop(0, _EPP // 16, grp,
                        (jnp.zeros((16,), jnp.int32),) * _CH)
    for k in range(_CH):
        pltpu.sync_copy(vstage.at[k], lv_hbm.at[pl.ds((k * _NW + w) * _CAP, _CAP)])
        pltpu.sync_copy(dstage.at[k], ld_hbm.at[pl.ds((k * _NW + w) * _CAP, _CAP)])
    cvec = jnp.zeros((16,), jnp.int32)
    for k in range(_CH):
        cvec = jnp.where(iota == k, cur[k], cvec)
    cbuf[...] = cvec
    pltpu.sync_copy(cbuf, cnt_hbm.at[pl.ds(w * 16, 16)])


@functools.partial(
    pl.kernel,
    out_type=jax.ShapeDtypeStruct((_M, _D), jnp.float32),
    mesh=_sc_mesh(),
    compiler_params=pltpu.CompilerParams(needs_layout_passes=False),
    scratch_types=[
        pltpu.VMEM((_CAP,), jnp.int32),
        pltpu.VMEM((_CAP,), jnp.int32),
        pltpu.VMEM((64, _D), jnp.float32),
        pltpu.VMEM((32, _D), jnp.float32),
        pltpu.VMEM((32,), jnp.int32),
        pltpu.VMEM((16,), jnp.int32),
        pltpu.VMEM_SHARED((_ACC1, _D), jnp.float32),
        pltpu.SemaphoreType.DMA,
    ],
)
def _s1(xv_hbm, lv_hbm, ld_hbm, cnt_hbm, z_hbm, xe_hbm,
        vbuf, dbuf, zbuf, rows0, dst0, cntv, acc, sem0):
    c = lax.axis_index("c")
    s = lax.axis_index("s")
    pltpu.sync_copy(z_hbm, zbuf)
    iota = lax.broadcasted_iota(jnp.int32, (16,), 0)
    for kk in range(_CH // _NC):
        k = c * (_CH // _NC) + kk
        lo = k * _MC
        for zz in range(7):
            pltpu.sync_copy(zbuf, acc.at[pl.ds(s * 504 + zz * 64, 64)])
        pltpu.sync_copy(zbuf.at[pl.ds(0, 56)], acc.at[pl.ds(s * 504 + 448, 56)])
        plsc.subcore_barrier()
        for li in range(2):
            w = s * 2 + li
            lidx = k * _NW + w
            pltpu.sync_copy(cnt_hbm.at[pl.ds(w * 16, 16)], cntv)
            cnt = jnp.max(jnp.where(iota == k, cntv[...], jnp.int32(0)))
            nt = lax.div(cnt + 31, jnp.int32(32))
            pltpu.sync_copy(lv_hbm.at[pl.ds(lidx * _CAP, _CAP)], vbuf)
            pltpu.sync_copy(ld_hbm.at[pl.ds(lidx * _CAP, _CAP)], dbuf)

            def tile(t, carry):
                b0 = t * 32
                for i in range(2):
                    dst0[pl.ds(i * 16, 16)] = dbuf[pl.ds(b0 + i * 16, 16)]
                pltpu.async_copy(
                    xv_hbm.at[vbuf.at[pl.ds(b0, 32)]], rows0, sem0).wait()
                pltpu.sync_copy(rows0, acc.at[dst0], add=True)
                return carry

            lax.fori_loop(0, nt, tile, 0)
        plsc.subcore_barrier()

        @pl.when(s < _NS - 1)
        def _():
            pltpu.sync_copy(acc.at[pl.ds(s * 504, 504)],
                            xe_hbm.at[pl.ds(lo + s * 504, 504)])

        @pl.when(s == _NS - 1)
        def _():
            pltpu.sync_copy(acc.at[pl.ds(7560, 440)],
                            xe_hbm.at[pl.ds(lo + 7560, 440)])

        if kk < _CH // _NC - 1:
            plsc.subcore_barrier()


@functools.partial(
    pl.kernel,
    out_type=jax.ShapeDtypeStruct((_NC, _ACC2, _D), jnp.float32),
    mesh=_sc_mesh(),
    compiler_params=pltpu.CompilerParams(needs_layout_passes=False),
    scratch_types=[
        pltpu.VMEM((_PAD2,), jnp.int32),
        pltpu.VMEM((_PAD2,), jnp.int32),
        pltpu.VMEM((64, _D), jnp.float32),
        pltpu.VMEM((32,), jnp.int32),
        pltpu.VMEM_SHARED((_ACC2, _D), jnp.float32),
        pltpu.SemaphoreType.DMA,
    ],
)
def _s2(y_hbm, v_hbm, e_hbm, z_hbm, hv_hbm,
        vsl, esl, rows0, dst0, acc, sem0):
    c = lax.axis_index("c")
    s = lax.axis_index("s")
    base = (c * _NS + s) * _EPP
    pltpu.sync_copy(v_hbm.at[pl.ds(base, _EPP)], vsl.at[pl.ds(0, _EPP)])
    pltpu.sync_copy(e_hbm.at[pl.ds(base, _EPP)], esl.at[pl.ds(0, _EPP)])
    for i in range(7):
        vsl[pl.ds(_EPP + i * 16, 16)] = jnp.full((16,), _N, jnp.int32)
        esl[pl.ds(_EPP + i * 16, 16)] = jnp.zeros((16,), jnp.int32)
    pltpu.sync_copy(z_hbm, rows0)
    zoff = s * 632
    for zz in range(9):
        pltpu.sync_copy(rows0, acc.at[pl.ds(zoff + zz * 64, 64)])
    pltpu.sync_copy(rows0.at[pl.ds(0, 56)], acc.at[pl.ds(zoff + 576, 56)])
    plsc.subcore_barrier()

    def tile(t, carry):
        b0 = t * 32
        for i in range(2):
            dst0[pl.ds(i * 16, 16)] = vsl[pl.ds(b0 + i * 16, 16)]
        pltpu.async_copy(y_hbm.at[esl.at[pl.ds(b0, 32)]],
                         rows0.at[pl.ds(0, 32)], sem0).wait()
        pltpu.sync_copy(rows0.at[pl.ds(0, 32)], acc.at[dst0], add=True)
        return carry

    lax.fori_loop(0, _PAD2 // 32, tile, 0)
    plsc.subcore_barrier()
    pltpu.sync_copy(acc.at[pl.ds(s * 632, 632)],
                    hv_hbm.at[c, pl.ds(s * 632, 632)])


def _dot(a, b):
    return jnp.dot(a, b, preferred_element_type=jnp.float32)


def _dotx(a, b):
    return jnp.dot(a, b, preferred_element_type=jnp.float32,
                   precision=lax.Precision.HIGHEST)


def _enc_body(xp_ref, embp_ref, w_ref, b_ref, out_ref):
    iota = lax.broadcasted_iota(jnp.int32, (1, _D), 1)
    h = jnp.zeros((1000, _D), jnp.float32)
    for col in range(9):
        oh = (xp_ref[:, col:col + 1] == iota).astype(jnp.float32)
        h = h + _dotx(oh, embp_ref[col])
    out_ref[...] = _dot(h, w_ref[...]) + b_ref[...]


def _mid0_body(xe_ref, ea_ref, bond_ref, w2_ref, b2_ref, w4_ref, b4_ref,
               y_ref, en_ref):
    iota = lax.broadcasted_iota(jnp.int32, (1, 8), 1)
    oh = (ea_ref[:, 0:1] == iota).astype(jnp.float32)
    e0 = _dotx(oh, bond_ref[...])
    y = _dot(xe_ref[...] + e0, w2_ref[...]) + b2_ref[...]
    y_ref[...] = y
    en_ref[...] = jnp.maximum(_dot(y, w4_ref[...]) + b4_ref[...], 0.0)


def _mid_body(xe_ref, e_ref, w2_ref, b2_ref, w4_ref, b4_ref, y_ref, en_ref):
    y = _dot(xe_ref[...] + e_ref[...], w2_ref[...]) + b2_ref[...]
    y_ref[...] = y
    en_ref[...] = jnp.maximum(_dot(y, w4_ref[...]) + b4_ref[...], 0.0)


def _midl_body(xe_ref, e_ref, w2_ref, b2_ref, y_ref):
    y_ref[...] = _dot(xe_ref[...] + e_ref[...], w2_ref[...]) + b2_ref[...]


def _p1_body(a_ref, b_ref, w3_ref, b3_ref, hn_ref, ps_ref, pq_ref):
    hv = a_ref[0] + b_ref[0]
    hn = _dot(hv, w3_ref[...]) + b3_ref[...]
    hn_ref[...] = hn
    ps_ref[...] = jnp.sum(hn, axis=0, keepdims=True)[None]
    pq_ref[...] = jnp.sum(hn * hn, axis=0, keepdims=True)[None]


def _bn(hn_ref, ps_ref, pq_ref, g_ref, bb_ref):
    mu = jnp.sum(ps_ref[...], axis=0) / _N
    var = jnp.sum(pq_ref[...], axis=0) / _N - mu * mu
    return (hn_ref[...] - mu) * lax.rsqrt(var + 1e-5) * g_ref[...] + bb_ref[...]


def _p3_body(hn_ref, ps_ref, pq_ref, g_ref, bb_ref, w1_ref, b1_ref, out_ref):
    h = jnp.maximum(_bn(hn_ref, ps_ref, pq_ref, g_ref, bb_ref), 0.0)
    out_ref[...] = _dot(h, w1_ref[...]) + b1_ref[...]


def _p3l_body(hn_ref, ps_ref, pq_ref, g_ref, bb_ref, bt_ref, pool_ref):
    h = _bn(hn_ref, ps_ref, pq_ref, g_ref, bb_ref)
    iota = lax.broadcasted_iota(jnp.int32, (1, _B), 1)
    oh = (bt_ref[:, 0:1] == iota).astype(jnp.float32)
    blk = lax.dot_general(oh, h, (((0,), (0,)), ((), ())),
                          preferred_element_type=jnp.float32,
                          precision=lax.Precision.HIGHEST)

    @pl.when(pl.program_id(0) == 0)
    def _():
        pool_ref[...] = jnp.zeros_like(pool_ref)

    pool_ref[...] += blk


def _f_body(p_ref, w1_ref, b1_ref, w2_ref, b2_ref, o_ref):
    t = jnp.maximum(_dot(p_ref[...], w1_ref[...]) + b1_ref[...], 0.0)
    o_ref[...] = _dot(t, w2_ref[...]) + b2_ref[...]


def _full(shape):
    zeros = (0,) * len(shape)
    return pl.BlockSpec(shape, lambda i, z=zeros: z)


def kernel(x, edge_index0, edge_index1, edge_attr, n_e, batch, atom_emb,
           bond_emb, Ww, Wb, bn_g, bn_b, out_w1, out_b1, out_w2, out_b2):
    f32 = jnp.float32
    xp = jnp.pad(x.astype(jnp.int32), ((0, 0), (0, _D - 9)))
    embp = jnp.pad(atom_emb, ((0, 0), (0, _D - 119), (0, 0)))
    bondp = jnp.pad(bond_emb, ((0, 2), (0, 0)))
    ea8 = jnp.pad(edge_attr.astype(jnp.int32), ((0, 0), (0, 7)),
                  constant_values=99)
    bt8 = jnp.pad(batch.astype(jnp.int32)[:, None], ((0, 0), (0, 7)),
                  constant_values=999)
    v_idx = edge_index0.astype(jnp.int32)
    e_idx = edge_index1.astype(jnp.int32)
    zer = jnp.zeros((_ZR, _D), f32)
    dumv = jnp.zeros((_CAP,), jnp.int32)
    dumd = jnp.full((_CAP,), _MC, jnp.int32)
    Wbr = Wb[:, :, None, :]
    bng = bn_g[:, None, :]
    bnb = bn_b[:, None, :]

    lv, ld, cnts = _prep(v_idx, e_idx, dumv, dumd)

    xv = pl.pallas_call(
        _enc_body,
        grid=(10,),
        in_specs=[
            pl.BlockSpec((1000, _D), lambda i: (i, 0)),
            _full((9, _D, _D)),
            _full((_D, _D)),
            _full((1, _D)),
        ],
        out_specs=pl.BlockSpec((1000, _D), lambda i: (i, 0)),
        out_shape=jax.ShapeDtypeStruct((_N, _D), f32),
    )(xp, embp, Ww[0, 0], Wbr[0, 0])

    e_cur = None
    pooled = None
    for l in range(_L):
        xe = _s1(xv, lv, ld, cnts, zer)
        xe_spec = pl.BlockSpec((512, _D), lambda i: (i, 0))
        if l == 0:
            y, e_cur = pl.pallas_call(
                _mid0_body,
                grid=(125,),
                in_specs=[
                    xe_spec,
                    pl.BlockSpec((512, 8), lambda i: (i, 0)),
                    _full((8, _D)), _full((_D, _D)), _full((1, _D)),
                    _full((_D, _D)), _full((1, _D)),
                ],
                out_specs=[xe_spec, xe_spec],
                out_shape=[jax.ShapeDtypeStruct((_M, _D), f32)] * 2,
            )(xe, ea8, bondp, Ww[0, 1], Wbr[0, 1], Ww[0, 3], Wbr[0, 3])
        elif l == 1:
            y, e_cur = pl.pallas_call(
                _mid_body,
                grid=(125,),
                in_specs=[xe_spec, xe_spec, _full((_D, _D)), _full((1, _D)),
                          _full((_D, _D)), _full((1, _D))],
                out_specs=[xe_spec, xe_spec],
                out_shape=[jax.ShapeDtypeStruct((_M, _D), f32)] * 2,
            )(xe, e_cur, Ww[1, 1], Wbr[1, 1], Ww[1, 3], Wbr[1, 3])
        else:
            y = pl.pallas_call(
                _midl_body,
                grid=(125,),
                in_specs=[xe_spec, xe_spec, _full((_D, _D)), _full((1, _D))],
                out_specs=xe_spec,
                out_shape=jax.ShapeDtypeStruct((_M, _D), f32),
            )(xe, e_cur, Ww[2, 1], Wbr[2, 1])

        hv2 = _s2(y, v_idx, e_idx, zer)
        hb = pl.BlockSpec((1000, _D), lambda i: (i, 0))
        rb = pl.BlockSpec((1, 1, _D), lambda i: (i, 0, 0))
        hn, ps, pq = pl.pallas_call(
            _p1_body,
            grid=(10,),
            in_specs=[
                pl.BlockSpec((1, 1000, _D), lambda i: (0, i, 0)),
                pl.BlockSpec((1, 1000, _D), lambda i: (1, i, 0)),
                _full((_D, _D)), _full((1, _D)),
            ],
            out_specs=[hb, rb, rb],
            out_shape=[jax.ShapeDtypeStruct((_N, _D), f32),
                       jax.ShapeDtypeStruct((10, 1, _D), f32),
                       jax.ShapeDtypeStruct((10, 1, _D), f32)],
        )(hv2, hv2, Ww[l, 2], Wbr[l, 2])

        if l < _L - 1:
            xv = pl.pallas_call(
                _p3_body,
                grid=(10,),
                in_specs=[hb, _full((10, 1, _D)), _full((10, 1, _D)),
                          _full((1, _D)), _full((1, _D)),
                          _full((_D, _D)), _full((1, _D))],
                out_specs=hb,
                out_shape=jax.ShapeDtypeStruct((_N, _D), f32),
            )(hn, ps, pq, bng[l], bnb[l], Ww[l + 1, 0], Wbr[l + 1, 0])
        else:
            pooled = pl.pallas_call(
                _p3l_body,
                grid=(10,),
                in_specs=[hb, _full((10, 1, _D)), _full((10, 1, _D)),
                          _full((1, _D)), _full((1, _D)),
                          pl.BlockSpec((1000, 8), lambda i: (i, 0))],
                out_specs=_full((_B, _D)),
                out_shape=jax.ShapeDtypeStruct((_B, _D), f32),
            )(hn, ps, pq, bng[2], bnb[2], bt8)

    w2p = jnp.pad(out_w2, ((0, 0), (0, 7)))
    b2p = jnp.pad(out_b2, (0, 7))[None, :]
    res = pl.pallas_call(
        _f_body,
        grid=(1,),
        in_specs=[_full((_B, _D)), _full((_D, _D)), _full((1, _D)),
                  _full((_D, 8)), _full((1, 8))],
        out_specs=_full((_B, 8)),
        out_shape=jax.ShapeDtypeStruct((_B, 8), f32),
    )(pooled, out_w1, out_b1[None, :], w2p, b2p)
    return res[:, 0]


# speedup vs baseline: 1.0375x; 1.0375x over previous
import functools

import jax
import jax.numpy as jnp
from jax import lax
from jax.experimental import pallas as pl
from jax.experimental.pallas import tpu as pltpu
from jax.experimental.pallas import tpu_sc as plsc

_N = 10000
_NNZ = 320000
_M = 64000
_B = 128
_D = 128
_L = 3

_NC = 2
_NS = 16
_NW = _NC * _NS

_CH = 8
_MC = 8000
_ACC1 = 8064
_CAP = 2048
_EPP = _NNZ // _NW

_ACC2 = 10112
_PAD2 = 10112

_ZR = 64


def _sc_mesh():
    return plsc.VectorSubcoreMesh(core_axis_name="c", subcore_axis_name="s",
                                  num_cores=_NC, num_subcores=_NS)


@functools.partial(
    pl.kernel,
    out_type=[jax.ShapeDtypeStruct((_CH * _NW * _CAP,), jnp.int32),
              jax.ShapeDtypeStruct((_CH * _NW * _CAP,), jnp.int32),
              jax.ShapeDtypeStruct((_NW * 16,), jnp.int32)],
    mesh=_sc_mesh(),
    compiler_params=pltpu.CompilerParams(needs_layout_passes=False),
    scratch_types=[
        pltpu.VMEM((_EPP,), jnp.int32),
        pltpu.VMEM((_EPP,), jnp.int32),
        pltpu.VMEM((_CH, _CAP), jnp.int32),
        pltpu.VMEM((_CH, _CAP), jnp.int32),
        pltpu.VMEM((16,), jnp.int32),
    ],
)
def _prep(v_hbm, e_hbm, dumv_hbm, dumd_hbm, lv_hbm, ld_hbm, cnt_hbm,
          vsl, esl, vstage, dstage, cbuf):
    c = lax.axis_index("c")
    s = lax.axis_index("s")
    w = c * _NS + s
    pltpu.sync_copy(v_hbm.at[pl.ds(w * _EPP, _EPP)], vsl)
    pltpu.sync_copy(e_hbm.at[pl.ds(w * _EPP, _EPP)], esl)
    for k in range(_CH):
        pltpu.sync_copy(dumv_hbm, vstage.at[k])
        pltpu.sync_copy(dumd_hbm, dstage.at[k])
    iota = lax.broadcasted_iota(jnp.int32, (16,), 0)

    def grp(g, cur):
        off = g * 16
        e16 = esl[pl.ds(off, 16)]
        v16 = vsl[pl.ds(off, 16)]
        c16 = lax.div(e16, jnp.int32(_MC))
        d16 = e16 - c16 * _MC
        out = []
        for k in range(_CH):
            m = c16 == jnp.int32(k)
            cs = plsc.cumsum(jnp.where(m, jnp.int32(1), jnp.int32(0)))
            pos = cur[k] + cs - 1
            mm = m & (pos < _CAP)
            kvec = jnp.full((16,), k, jnp.int32)
            plsc.store_scatter(vstage, [kvec, pos], v16, mask=mm)
            plsc.store_scatter(dstage, [kvec, pos], d16, mask=mm)
            out.append(cur[k] + plsc.all_reduce_population_count(m))
        return tuple(out)

    cur = lax.fori_loop(0, _EPP // 16, grp,
                        (jnp.zeros((16,), jnp.int32),) * _CH)
    for k in range(_CH):
        pltpu.sync_copy(vstage.at[k], lv_hbm.at[pl.ds((k * _NW + w) * _CAP, _CAP)])
        pltpu.sync_copy(dstage.at[k], ld_hbm.at[pl.ds((k * _NW + w) * _CAP, _CAP)])
    cvec = jnp.zeros((16,), jnp.int32)
    for k in range(_CH):
        cvec = jnp.where(iota == k, cur[k], cvec)
    cbuf[...] = cvec
    pltpu.sync_copy(cbuf, cnt_hbm.at[pl.ds(w * 16, 16)])


@functools.partial(
    pl.kernel,
    out_type=jax.ShapeDtypeStruct((_M, _D), jnp.float32),
    mesh=_sc_mesh(),
    compiler_params=pltpu.CompilerParams(needs_layout_passes=False),
    scratch_types=[
        pltpu.VMEM((_CAP,), jnp.int32),
        pltpu.VMEM((_CAP,), jnp.int32),
        pltpu.VMEM((64, _D), jnp.float32),
        pltpu.VMEM((64, _D), jnp.float32),
        pltpu.VMEM((64,), jnp.int32),
        pltpu.VMEM((16,), jnp.int32),
        pltpu.VMEM_SHARED((_ACC1, _D), jnp.float32),
        pltpu.SemaphoreType.DMA,
    ],
)
def _s1(xv_hbm, lv_hbm, ld_hbm, cnt_hbm, z_hbm, xe_hbm,
        vbuf, dbuf, zbuf, rows0, dst0, cntv, acc, sem0):
    c = lax.axis_index("c")
    s = lax.axis_index("s")
    pltpu.sync_copy(z_hbm, zbuf)
    iota = lax.broadcasted_iota(jnp.int32, (16,), 0)
    for kk in range(_CH // _NC):
        k = c * (_CH // _NC) + kk
        lo = k * _MC
        for zz in range(7):
            pltpu.sync_copy(zbuf, acc.at[pl.ds(s * 504 + zz * 64, 64)])
        pltpu.sync_copy(zbuf.at[pl.ds(0, 56)], acc.at[pl.ds(s * 504 + 448, 56)])
        plsc.subcore_barrier()
        for li in range(2):
            w = s * 2 + li
            lidx = k * _NW + w
            pltpu.sync_copy(cnt_hbm.at[pl.ds(w * 16, 16)], cntv)
            cnt = jnp.max(jnp.where(iota == k, cntv[...], jnp.int32(0)))
            nt = lax.div(cnt + 63, jnp.int32(64))
            pltpu.sync_copy(lv_hbm.at[pl.ds(lidx * _CAP, _CAP)], vbuf)
            pltpu.sync_copy(ld_hbm.at[pl.ds(lidx * _CAP, _CAP)], dbuf)

            def tile(t, carry):
                b0 = t * 64
                for i in range(4):
                    dst0[pl.ds(i * 16, 16)] = dbuf[pl.ds(b0 + i * 16, 16)]
                pltpu.async_copy(
                    xv_hbm.at[vbuf.at[pl.ds(b0, 64)]], rows0, sem0).wait()
                pltpu.sync_copy(rows0, acc.at[dst0], add=True)
                return carry

            lax.fori_loop(0, nt, tile, 0)
        plsc.subcore_barrier()

        @pl.when(s < _NS - 1)
        def _():
            pltpu.sync_copy(acc.at[pl.ds(s * 504, 504)],
                            xe_hbm.at[pl.ds(lo + s * 504, 504)])

        @pl.when(s == _NS - 1)
        def _():
            pltpu.sync_copy(acc.at[pl.ds(7560, 440)],
                            xe_hbm.at[pl.ds(lo + 7560, 440)])

        if kk < _CH // _NC - 1:
            plsc.subcore_barrier()


@functools.partial(
    pl.kernel,
    out_type=jax.ShapeDtypeStruct((_NC, _ACC2, _D), jnp.float32),
    mesh=_sc_mesh(),
    compiler_params=pltpu.CompilerParams(needs_layout_passes=False),
    scratch_types=[
        pltpu.VMEM((_PAD2,), jnp.int32),
        pltpu.VMEM((_PAD2,), jnp.int32),
        pltpu.VMEM((64, _D), jnp.float32),
        pltpu.VMEM((64,), jnp.int32),
        pltpu.VMEM_SHARED((_ACC2, _D), jnp.float32),
        pltpu.SemaphoreType.DMA,
    ],
)
def _s2(y_hbm, v_hbm, e_hbm, z_hbm, hv_hbm,
        vsl, esl, rows0, dst0, acc, sem0):
    c = lax.axis_index("c")
    s = lax.axis_index("s")
    base = (c * _NS + s) * _EPP
    pltpu.sync_copy(v_hbm.at[pl.ds(base, _EPP)], vsl.at[pl.ds(0, _EPP)])
    pltpu.sync_copy(e_hbm.at[pl.ds(base, _EPP)], esl.at[pl.ds(0, _EPP)])
    for i in range(7):
        vsl[pl.ds(_EPP + i * 16, 16)] = jnp.full((16,), _N, jnp.int32)
        esl[pl.ds(_EPP + i * 16, 16)] = jnp.zeros((16,), jnp.int32)
    pltpu.sync_copy(z_hbm, rows0)
    zoff = s * 632
    for zz in range(9):
        pltpu.sync_copy(rows0, acc.at[pl.ds(zoff + zz * 64, 64)])
    pltpu.sync_copy(rows0.at[pl.ds(0, 56)], acc.at[pl.ds(zoff + 576, 56)])
    plsc.subcore_barrier()

    def tile(t, carry):
        b0 = t * 64
        for i in range(4):
            dst0[pl.ds(i * 16, 16)] = vsl[pl.ds(b0 + i * 16, 16)]
        pltpu.async_copy(y_hbm.at[esl.at[pl.ds(b0, 64)]], rows0, sem0).wait()
        pltpu.sync_copy(rows0, acc.at[dst0], add=True)
        return carry

    lax.fori_loop(0, _PAD2 // 64, tile, 0)
    plsc.subcore_barrier()
    pltpu.sync_copy(acc.at[pl.ds(s * 632, 632)],
                    hv_hbm.at[c, pl.ds(s * 632, 632)])


def _dot(a, b):
    return jnp.dot(a, b, preferred_element_type=jnp.float32)


def _dotx(a, b):
    return jnp.dot(a, b, preferred_element_type=jnp.float32,
                   precision=lax.Precision.HIGHEST)


def _enc_body(xp_ref, embp_ref, w_ref, b_ref, out_ref):
    iota = lax.broadcasted_iota(jnp.int32, (1, _D), 1)
    h = jnp.zeros((1000, _D), jnp.float32)
    for col in range(9):
        oh = (xp_ref[:, col:col + 1] == iota).astype(jnp.float32)
        h = h + _dotx(oh, embp_ref[col])
    out_ref[...] = _dot(h, w_ref[...]) + b_ref[...]


def _mid0_body(xe_ref, ea_ref, bond_ref, w2_ref, b2_ref, y_ref):
    iota = lax.broadcasted_iota(jnp.int32, (1, 8), 1)
    oh = (ea_ref[:, 0:1] == iota).astype(jnp.float32)
    e0 = _dotx(oh, bond_ref[...])
    y_ref[...] = _dot(xe_ref[...] + e0, w2_ref[...]) + b2_ref[...]


def _midl_body(xe_ref, e_ref, w2_ref, b2_ref, y_ref):
    y_ref[...] = _dot(xe_ref[...] + e_ref[...], w2_ref[...]) + b2_ref[...]


def _e_body(y_ref, w4_ref, b4_ref, en_ref):
    en_ref[...] = jnp.maximum(_dot(y_ref[...], w4_ref[...]) + b4_ref[...], 0.0)


def _p1_body(a_ref, b_ref, w3_ref, b3_ref, hn_ref, ps_ref, pq_ref):
    hv = a_ref[0] + b_ref[0]
    hn = _dot(hv, w3_ref[...]) + b3_ref[...]
    hn_ref[...] = hn
    ps_ref[...] = jnp.sum(hn, axis=0, keepdims=True)[None]
    pq_ref[...] = jnp.sum(hn * hn, axis=0, keepdims=True)[None]


def _bn(hn_ref, ps_ref, pq_ref, g_ref, bb_ref):
    mu = jnp.sum(ps_ref[...], axis=0) / _N
    var = jnp.sum(pq_ref[...], axis=0) / _N - mu * mu
    return (hn_ref[...] - mu) * lax.rsqrt(var + 1e-5) * g_ref[...] + bb_ref[...]


def _p3_body(hn_ref, ps_ref, pq_ref, g_ref, bb_ref, w1_ref, b1_ref, out_ref):
    h = jnp.maximum(_bn(hn_ref, ps_ref, pq_ref, g_ref, bb_ref), 0.0)
    out_ref[...] = _dot(h, w1_ref[...]) + b1_ref[...]


def _p3l_body(hn_ref, ps_ref, pq_ref, g_ref, bb_ref, bt_ref, pool_ref):
    h = _bn(hn_ref, ps_ref, pq_ref, g_ref, bb_ref)
    iota = lax.broadcasted_iota(jnp.int32, (1, _B), 1)
    oh = (bt_ref[:, 0:1] == iota).astype(jnp.float32)
    blk = lax.dot_general(oh, h, (((0,), (0,)), ((), ())),
                          preferred_element_type=jnp.float32,
                          precision=lax.Precision.HIGHEST)

    @pl.when(pl.program_id(0) == 0)
    def _():
        pool_ref[...] = jnp.zeros_like(pool_ref)

    pool_ref[...] += blk


def _f_body(p_ref, w1_ref, b1_ref, w2_ref, b2_ref, o_ref):
    t = jnp.maximum(_dot(p_ref[...], w1_ref[...]) + b1_ref[...], 0.0)
    o_ref[...] = _dot(t, w2_ref[...]) + b2_ref[...]


def _full(shape):
    zeros = (0,) * len(shape)
    return pl.BlockSpec(shape, lambda i, z=zeros: z)


def kernel(x, edge_index0, edge_index1, edge_attr, n_e, batch, atom_emb,
           bond_emb, Ww, Wb, bn_g, bn_b, out_w1, out_b1, out_w2, out_b2):
    f32 = jnp.float32
    xp = jnp.pad(x.astype(jnp.int32), ((0, 0), (0, _D - 9)))
    embp = jnp.pad(atom_emb, ((0, 0), (0, _D - 119), (0, 0)))
    bondp = jnp.pad(bond_emb, ((0, 2), (0, 0)))
    ea8 = jnp.pad(edge_attr.astype(jnp.int32), ((0, 0), (0, 7)),
                  constant_values=99)
    bt8 = jnp.pad(batch.astype(jnp.int32)[:, None], ((0, 0), (0, 7)),
                  constant_values=999)
    v_idx = edge_index0.astype(jnp.int32)
    e_idx = edge_index1.astype(jnp.int32)
    zer = jnp.zeros((_ZR, _D), f32)
    dumv = jnp.zeros((_CAP,), jnp.int32)
    dumd = jnp.full((_CAP,), _MC, jnp.int32)
    Wbr = Wb[:, :, None, :]
    bng = bn_g[:, None, :]
    bnb = bn_b[:, None, :]

    lv, ld, cnts = _prep(v_idx, e_idx, dumv, dumd)

    xv = pl.pallas_call(
        _enc_body,
        grid=(10,),
        in_specs=[
            pl.BlockSpec((1000, _D), lambda i: (i, 0)),
            _full((9, _D, _D)),
            _full((_D, _D)),
            _full((1, _D)),
        ],
        out_specs=pl.BlockSpec((1000, _D), lambda i: (i, 0)),
        out_shape=jax.ShapeDtypeStruct((_N, _D), f32),
    )(xp, embp, Ww[0, 0], Wbr[0, 0])

    e_cur = None
    pooled = None
    for l in range(_L):
        xe = _s1(xv, lv, ld, cnts, zer)
        xe_spec = pl.BlockSpec((512, _D), lambda i: (i, 0))
        if l == 0:
            y = pl.pallas_call(
                _mid0_body,
                grid=(125,),
                in_specs=[
                    xe_spec,
                    pl.BlockSpec((512, 8), lambda i: (i, 0)),
                    _full((8, _D)), _full((_D, _D)), _full((1, _D)),
                ],
                out_specs=xe_spec,
                out_shape=jax.ShapeDtypeStruct((_M, _D), f32),
            )(xe, ea8, bondp, Ww[0, 1], Wbr[0, 1])
        else:
            y = pl.pallas_call(
                _midl_body,
                grid=(125,),
                in_specs=[xe_spec, xe_spec, _full((_D, _D)), _full((1, _D))],
                out_specs=xe_spec,
                out_shape=jax.ShapeDtypeStruct((_M, _D), f32),
            )(xe, e_cur, Ww[l, 1], Wbr[l, 1])

        hv2 = _s2(y, v_idx, e_idx, zer)
        if l < _L - 1:
            e_cur = pl.pallas_call(
                _e_body,
                grid=(125,),
                in_specs=[xe_spec, _full((_D, _D)), _full((1, _D))],
                out_specs=xe_spec,
                out_shape=jax.ShapeDtypeStruct((_M, _D), f32),
            )(y, Ww[l, 3], Wbr[l, 3])
        hb = pl.BlockSpec((1000, _D), lambda i: (i, 0))
        rb = pl.BlockSpec((1, 1, _D), lambda i: (i, 0, 0))
        hn, ps, pq = pl.pallas_call(
            _p1_body,
            grid=(10,),
            in_specs=[
                pl.BlockSpec((1, 1000, _D), lambda i: (0, i, 0)),
                pl.BlockSpec((1, 1000, _D), lambda i: (1, i, 0)),
                _full((_D, _D)), _full((1, _D)),
            ],
            out_specs=[hb, rb, rb],
            out_shape=[jax.ShapeDtypeStruct((_N, _D), f32),
                       jax.ShapeDtypeStruct((10, 1, _D), f32),
                       jax.ShapeDtypeStruct((10, 1, _D), f32)],
        )(hv2, hv2, Ww[l, 2], Wbr[l, 2])

        if l < _L - 1:
            xv = pl.pallas_call(
                _p3_body,
                grid=(10,),
                in_specs=[hb, _full((10, 1, _D)), _full((10, 1, _D)),
                          _full((1, _D)), _full((1, _D)),
                          _full((_D, _D)), _full((1, _D))],
                out_specs=hb,
                out_shape=jax.ShapeDtypeStruct((_N, _D), f32),
            )(hn, ps, pq, bng[l], bnb[l], Ww[l + 1, 0], Wbr[l + 1, 0])
        else:
            pooled = pl.pallas_call(
                _p3l_body,
                grid=(10,),
                in_specs=[hb, _full((10, 1, _D)), _full((10, 1, _D)),
                          _full((1, _D)), _full((1, _D)),
                          pl.BlockSpec((1000, 8), lambda i: (i, 0))],
                out_specs=_full((_B, _D)),
                out_shape=jax.ShapeDtypeStruct((_B, _D), f32),
            )(hn, ps, pq, bng[2], bnb[2], bt8)

    w2p = jnp.pad(out_w2, ((0, 0), (0, 7)))
    b2p = jnp.pad(out_b2, (0, 7))[None, :]
    res = pl.pallas_call(
        _f_body,
        grid=(1,),
        in_specs=[_full((_B, _D)), _full((_D, _D)), _full((1, _D)),
                  _full((_D, 8)), _full((1, 8))],
        out_specs=_full((_B, 8)),
        out_shape=jax.ShapeDtypeStruct((_B, 8), f32),
    )(pooled, out_w1, out_b1[None, :], w2p, b2p)
    return res[:, 0]
